# baseline (speedup 1.0000x reference)
import jax
import jax.numpy as jnp
from jax.experimental import pallas as pl
from jax.experimental.pallas import tpu as pltpu

_N = 16384
_DIM = 2048
_E = 64
_TILE = 1024


def _top2(logits, b):
    logits = logits + b
    colf = jax.lax.broadcasted_iota(
        jnp.int32, logits.shape, 1).astype(jnp.float32)
    m1 = jnp.max(logits, axis=1, keepdims=True)
    i1f = jnp.min(jnp.where(logits == m1, colf, float(_E)),
                  axis=1, keepdims=True)
    masked = jnp.where(colf == i1f, -jnp.inf, logits)
    m2 = jnp.max(masked, axis=1, keepdims=True)
    i2f = jnp.min(jnp.where(masked == m2, colf, float(_E)),
                  axis=1, keepdims=True)
    denom = jnp.sum(jnp.exp(logits - m1), axis=1, keepdims=True)
    v1 = 1.0 / denom
    v2 = jnp.exp(m2 - m1) * v1
    vals = jnp.concatenate([v1, v2], axis=1)
    idx = jnp.concatenate([i1f, i2f], axis=1).astype(jnp.int32)
    return vals, idx


def _gate_tile(xa_ref, xb_ref, w_ref, b_ref, vals_ref, idx_ref):
    w = w_ref[...]
    b = b_ref[...]
    dn = (((1,), (1,)), ((), ()))
    la = jax.lax.dot_general(xa_ref[...], w, dn,
                             preferred_element_type=jnp.float32)
    lb = jax.lax.dot_general(xb_ref[...], w, dn,
                             preferred_element_type=jnp.float32)
    va, ia = _top2(la, b)
    vb, ib = _top2(lb, b)
    vals_ref[...] = jnp.concatenate([va, vb], axis=0)
    idx_ref[...] = jnp.concatenate([ia, ib], axis=0)


def kernel(x, weight, bias):
    n = x.shape[0]
    grid = (n // (2 * _TILE),)
    vals, idx = pl.pallas_call(
        _gate_tile,
        grid=grid,
        in_specs=[
            pl.BlockSpec((_TILE, _DIM), lambda i: (2 * i, 0)),
            pl.BlockSpec((_TILE, _DIM), lambda i: (2 * i + 1, 0)),
            pl.BlockSpec((_E, _DIM), lambda i: (0, 0)),
            pl.BlockSpec((1, _E), lambda i: (0, 0)),
        ],
        out_specs=[
            pl.BlockSpec((2 * _TILE, 2), lambda i: (i, 0)),
            pl.BlockSpec((2 * _TILE, 2), lambda i: (i, 0)),
        ],
        out_shape=[
            jax.ShapeDtypeStruct((n, 2), jnp.float32),
            jax.ShapeDtypeStruct((n, 2), jnp.int32),
        ],
        compiler_params=pltpu.CompilerParams(
            dimension_semantics=("arbitrary",)),
    )(x, x, weight, bias.reshape(1, _E))
    return vals, idx


# speedup vs baseline: 1.3319x; 1.3319x over previous
import jax
import jax.numpy as jnp
from jax.experimental import pallas as pl
from jax.experimental.pallas import tpu as pltpu

_N = 16384
_DIM = 2048
_E = 64
_TILE = 2048


def _gate_tile(x_ref, w_ref, b_ref, vals_ref, idx_ref):
    x = x_ref[...]
    w = w_ref[...]
    logits = jax.lax.dot_general(
        w, x, (((1,), (1,)), ((), ())), preferred_element_type=jnp.float32)
    logits = logits + b_ref[...]

    rowf = jax.lax.broadcasted_iota(
        jnp.int32, logits.shape, 0).astype(jnp.float32)

    m1 = jnp.max(logits, axis=0, keepdims=True)
    i1f = jnp.min(jnp.where(logits == m1, rowf, float(_E)),
                  axis=0, keepdims=True)

    masked = jnp.where(rowf == i1f, -jnp.inf, logits)
    m2 = jnp.max(masked, axis=0, keepdims=True)
    i2f = jnp.min(jnp.where(masked == m2, rowf, float(_E)),
                  axis=0, keepdims=True)

    denom = jnp.sum(jnp.exp(logits - m1), axis=0, keepdims=True)
    v1 = 1.0 / denom
    v2 = jnp.exp(m2 - m1) * v1

    vals_ref[...] = jnp.concatenate([v1, v2], axis=0)
    idx_ref[...] = jnp.concatenate([i1f, i2f], axis=0).astype(jnp.int32)


def kernel(x, weight, bias):
    n = x.shape[0]
    grid = (n // _TILE,)
    vals_t, idx_t = pl.pallas_call(
        _gate_tile,
        grid=grid,
        in_specs=[
            pl.BlockSpec((_TILE, _DIM), lambda i: (i, 0)),
            pl.BlockSpec((_E, _DIM), lambda i: (0, 0)),
            pl.BlockSpec((_E, 1), lambda i: (0, 0)),
        ],
        out_specs=[
            pl.BlockSpec((2, _TILE), lambda i: (0, i)),
            pl.BlockSpec((2, _TILE), lambda i: (0, i)),
        ],
        out_shape=[
            jax.ShapeDtypeStruct((2, n), jnp.float32),
            jax.ShapeDtypeStruct((2, n), jnp.int32),
        ],
        compiler_params=pltpu.CompilerParams(
            dimension_semantics=("arbitrary",)),
    )(x, weight, bias.reshape(_E, 1))
    return vals_t.T, idx_t.T


# speedup vs baseline: 1.3394x; 1.0056x over previous
import jax
import jax.numpy as jnp
from jax.experimental import pallas as pl
from jax.experimental.pallas import tpu as pltpu

_N = 16384
_DIM = 2048
_E = 64
_TILE = 1024


def _gate_tile(x_ref, w_ref, b_ref, vals_ref, idx_ref):
    x = x_ref[...]
    w = w_ref[...]
    logits = jax.lax.dot_general(
        w, x, (((1,), (1,)), ((), ())), preferred_element_type=jnp.float32)
    logits = logits + b_ref[...]

    rowf = jax.lax.broadcasted_iota(
        jnp.int32, logits.shape, 0).astype(jnp.float32)

    m1 = jnp.max(logits, axis=0, keepdims=True)
    i1f = jnp.min(jnp.where(logits == m1, rowf, float(_E)),
                  axis=0, keepdims=True)

    masked = jnp.where(rowf == i1f, -jnp.inf, logits)
    m2 = jnp.max(masked, axis=0, keepdims=True)
    i2f = jnp.min(jnp.where(masked == m2, rowf, float(_E)),
                  axis=0, keepdims=True)

    denom = jnp.sum(jnp.exp(logits - m1), axis=0, keepdims=True)
    v1 = 1.0 / denom
    v2 = jnp.exp(m2 - m1) * v1

    vals_ref[...] = jnp.concatenate([v1, v2], axis=0)
    idx_ref[...] = jnp.concatenate([i1f, i2f], axis=0).astype(jnp.int32)


def kernel(x, weight, bias):
    n = x.shape[0]
    grid = (n // _TILE,)
    vals_t, idx_t = pl.pallas_call(
        _gate_tile,
        grid=grid,
        in_specs=[
            pl.BlockSpec((_TILE, _DIM), lambda i: (i, 0)),
            pl.BlockSpec((_E, _DIM), lambda i: (0, 0)),
            pl.BlockSpec((_E, 1), lambda i: (0, 0)),
        ],
        out_specs=[
            pl.BlockSpec((2, _TILE), lambda i: (0, i)),
            pl.BlockSpec((2, _TILE), lambda i: (0, i)),
        ],
        out_shape=[
            jax.ShapeDtypeStruct((2, n), jnp.float32),
            jax.ShapeDtypeStruct((2, n), jnp.int32),
        ],
        compiler_params=pltpu.CompilerParams(
            dimension_semantics=("arbitrary",)),
    )(x, weight, bias.reshape(_E, 1))
    return vals_t.T, idx_t.T
